# Initial kernel scaffold; baseline (speedup 1.0000x reference)
import jax
import jax.numpy as jnp
from jax.experimental import pallas as pl


def kernel(token_ids, embedding_matrix):
    raise NotImplementedError("write your pallas kernel here")


# speedup vs baseline: 1.3067x; 1.3067x over previous
import functools

import jax
import jax.numpy as jnp
from jax import lax
from jax.experimental import pallas as pl
from jax.experimental.pallas import tpu as pltpu, tpu_sc as plsc

NUM_CORES = 2
NUM_SUBCORES = 16
NUM_WORKERS = NUM_CORES * NUM_SUBCORES
CHUNK = 128


@functools.partial(jax.jit, static_argnames=("n_chunks", "dim"))
def _sc_gather(table, idx3, *, n_chunks, dim):
    rows_total = NUM_WORKERS * n_chunks * CHUNK
    b_per_w = n_chunks * CHUNK
    mesh = plsc.VectorSubcoreMesh(core_axis_name="c", subcore_axis_name="s")

    @functools.partial(
        pl.kernel,
        out_type=jax.ShapeDtypeStruct((rows_total, dim), jnp.float32),
        mesh=mesh,
        scratch_types=[
            pltpu.VMEM((n_chunks, CHUNK), jnp.int32),
            pltpu.VMEM((CHUNK, dim), jnp.float32),
            pltpu.SemaphoreType.DMA,
        ],
        compiler_params=pltpu.CompilerParams(use_tc_tiling_on_sc=False),
    )
    def k(table_hbm, idx_hbm, out_hbm, idx_v, rows_v, sem):
        wid = lax.axis_index("s") * NUM_CORES + lax.axis_index("c")
        base = wid * b_per_w
        pltpu.sync_copy(idx_hbm.at[wid], idx_v)

        def body(j, carry):
            pltpu.async_copy(table_hbm.at[idx_v.at[j]], rows_v, sem).wait()
            pltpu.sync_copy(rows_v, out_hbm.at[pl.ds(base + j * CHUNK, CHUNK)])
            return carry

        lax.fori_loop(0, n_chunks, body, 0)

    return k(table, idx3)


def kernel(token_ids, embedding_matrix):
    orig_shape = token_ids.shape
    dim = embedding_matrix.shape[1]
    flat = token_ids.reshape(-1).astype(jnp.int32)
    total = flat.shape[0]
    assert total % (NUM_WORKERS * CHUNK) == 0
    n_chunks = total // (NUM_WORKERS * CHUNK)
    idx3 = flat.reshape(NUM_WORKERS, n_chunks, CHUNK)
    out = _sc_gather(embedding_matrix, idx3, n_chunks=n_chunks, dim=dim)
    return out.reshape(*orig_shape, dim)


# speedup vs baseline: 1.4941x; 1.1434x over previous
import functools

import jax
import jax.numpy as jnp
from jax import lax
from jax.experimental import pallas as pl
from jax.experimental.pallas import tpu as pltpu, tpu_sc as plsc

NUM_CORES = 2
NUM_SUBCORES = 16
NUM_WORKERS = NUM_CORES * NUM_SUBCORES
CHUNK = 128


GROUP = 10


@functools.partial(jax.jit, static_argnames=("n_chunks", "dim"))
def _sc_gather(table, idx3, *, n_chunks, dim):
    rows_total = NUM_WORKERS * n_chunks * CHUNK
    b_per_w = n_chunks * CHUNK
    n_groups = n_chunks // GROUP
    assert n_chunks % GROUP == 0 and n_groups % 2 == 0
    g_rows = GROUP * CHUNK
    mesh = plsc.VectorSubcoreMesh(core_axis_name="c", subcore_axis_name="s")

    @functools.partial(
        pl.kernel,
        out_type=jax.ShapeDtypeStruct((rows_total, dim), jnp.float32),
        mesh=mesh,
        scratch_types=[
            pltpu.VMEM((n_chunks, CHUNK), jnp.int32),
            pltpu.VMEM((2, g_rows, dim), jnp.float32),
            pltpu.SemaphoreType.DMA,
            pltpu.SemaphoreType.DMA,
        ],
        compiler_params=pltpu.CompilerParams(use_tc_tiling_on_sc=False),
    )
    def k(table_hbm, idx_hbm, out_hbm, idx_v, rows_v, sem_g, sem_o):
        wid = lax.axis_index("s") * NUM_CORES + lax.axis_index("c")
        base = wid * b_per_w
        pltpu.sync_copy(idx_hbm.at[wid], idx_v)

        def gather_group(g, b):
            copies = [
                pltpu.async_copy(
                    table_hbm.at[idx_v.at[g * GROUP + i]],
                    rows_v.at[b, pl.ds(i * CHUNK, CHUNK)],
                    sem_g,
                )
                for i in range(GROUP)
            ]
            for c in copies:
                c.wait()

        def out_start(g, b):
            return pltpu.async_copy(
                rows_v.at[b], out_hbm.at[pl.ds(base + g * g_rows, g_rows)], sem_o
            )

        def out_wait(b):
            pltpu.make_async_copy(
                rows_v.at[b], out_hbm.at[pl.ds(base, g_rows)], sem_o
            ).wait()

        for b in range(2):
            gather_group(b, b)
            out_start(b, b)

        def body(t, carry):
            for b in range(2):
                g = 2 + 2 * t + b
                out_wait(b)
                gather_group(g, b)
                out_start(g, b)
            return carry

        lax.fori_loop(0, (n_groups - 2) // 2, body, 0)
        for b in range(2):
            out_wait(b)

    return k(table, idx3)


def kernel(token_ids, embedding_matrix):
    orig_shape = token_ids.shape
    dim = embedding_matrix.shape[1]
    flat = token_ids.reshape(-1).astype(jnp.int32)
    total = flat.shape[0]
    assert total % (NUM_WORKERS * CHUNK) == 0
    n_chunks = total // (NUM_WORKERS * CHUNK)
    idx3 = flat.reshape(NUM_WORKERS, n_chunks, CHUNK)
    out = _sc_gather(embedding_matrix, idx3, n_chunks=n_chunks, dim=dim)
    return out.reshape(*orig_shape, dim)
